# Initial kernel scaffold; baseline (speedup 1.0000x reference)
import jax
import jax.numpy as jnp
from jax.experimental import pallas as pl


def kernel(x, edge_index, Wl1, Wr1, att1, b1, Wl2, Wr2, att2, b2):
    raise NotImplementedError("write your pallas kernel here")


# speedup vs baseline: 1.1065x; 1.1065x over previous
import jax
import jax.numpy as jnp
from jax.experimental import pallas as pl


def _gatv2_layer(x, src, dst, Wl, Wr, att, bias, concat):
    N = x.shape[0]
    H, C = att.shape
    xl = (x @ Wl).reshape(N, H, C)
    xr = (x @ Wr).reshape(N, H, C)
    e = jax.nn.leaky_relu(xl[src] + xr[dst], negative_slope=0.2)
    logits = (e * att[None, :, :]).sum(-1)
    expv = jnp.exp(logits)
    denom = jax.ops.segment_sum(expv, dst, num_segments=N)
    alpha = expv / denom[dst]
    out = jax.ops.segment_sum(xl[src] * alpha[..., None], dst, num_segments=N)
    if concat:
        out = out.reshape(N, H * C)
    else:
        out = out.mean(axis=1)
    return out + bias


def _tail_kernel(o_ref, out_ref, ls_ref):
    o = o_ref[...]
    out_ref[...] = o
    m = jnp.max(o, axis=1, keepdims=True)
    s = jnp.log(jnp.sum(jnp.exp(o - m), axis=1, keepdims=True))
    ls_ref[...] = o - m - s


def kernel(x, edge_index, Wl1, Wr1, att1, b1, Wl2, Wr2, att2, b2):
    N = x.shape[0]
    loop = jnp.arange(N, dtype=edge_index.dtype)
    src = jnp.concatenate([edge_index[0], loop])
    dst = jnp.concatenate([edge_index[1], loop])
    h = jnp.tanh(_gatv2_layer(x, src, dst, Wl1, Wr1, att1, b1, concat=True))
    o = _gatv2_layer(h, src, dst, Wl2, Wr2, att2, b2, concat=False)
    out, ls = pl.pallas_call(
        _tail_kernel,
        out_shape=(jax.ShapeDtypeStruct(o.shape, o.dtype),
                   jax.ShapeDtypeStruct(o.shape, o.dtype)),
    )(o)
    return (out, ls)


# speedup vs baseline: 16.1635x; 14.6071x over previous
import jax
import jax.numpy as jnp
from jax import lax
from jax.experimental import pallas as pl
from jax.experimental.pallas import tpu as pltpu
from jax.experimental.pallas import tpu_sc as plsc

N = 10000
NPAD = 10240
ROWS_PER_SUB = 640
E_RAW = 320000
E_TOT = E_RAW + N
BLK = 128
BLKA = 64
CHUNKS = 81
CHUNKS_A = 162
NW = 32
E_PAD = NW * CHUNKS * BLK
DUMMY = N

_f32 = jnp.float32
_i32 = jnp.int32


def _mesh():
    return plsc.VectorSubcoreMesh(
        core_axis_name="c", subcore_axis_name="s", num_cores=2, num_subcores=16)


def _iota16():
    return lax.iota(_i32, 16)


def _lane15():
    return lax.iota(_i32, 16) == 15


def _store_scalar(ref, idxs, vec):
    plsc.store_scatter(ref, [jnp.full((16,), i, _i32) for i in idxs], vec,
                       mask=_lane15())


def _l1a_body(t1, srcp, dstp, attr, z4,
              expv1, dpart,
              cidx, didx, xlr, attv, lbuf, e4, dsh, sg1):
    cid = lax.axis_index("c")
    sid = lax.axis_index("s")
    pltpu.sync_copy(z4, dsh.at[pl.ds(sid * ROWS_PER_SUB, ROWS_PER_SUB), :])
    pltpu.sync_copy(attr, attv)
    plsc.subcore_barrier()
    w = cid * 16 + sid
    base_w = w * (CHUNKS_A * BLKA)

    def chunk(c, carry):
        base = base_w + c * BLKA
        pltpu.sync_copy(srcp.at[pl.ds(base, BLKA)], cidx.at[pl.ds(0, BLKA)])
        pltpu.sync_copy(dstp.at[pl.ds(base, BLKA)], cidx.at[pl.ds(BLKA, BLKA)])
        pltpu.sync_copy(dstp.at[pl.ds(base, BLKA)], didx)
        for k in range(BLKA // 16):
            s = pl.ds(BLKA + k * 16, 16)
            cidx[s] = cidx[s] + NPAD
        pltpu.async_copy(t1.at[cidx], xlr, sg1).wait()

        def edge(e, carry2):
            for h in range(4):
                j0, j1 = 2 * h, 2 * h + 1
                u0 = xlr[e, pl.ds(j0 * 16, 16)] + xlr[BLKA + e, pl.ds(j0 * 16, 16)]
                u1 = xlr[e, pl.ds(j1 * 16, 16)] + xlr[BLKA + e, pl.ds(j1 * 16, 16)]
                t = (jnp.maximum(u0, 0.2 * u0) * attv[j0, :]
                     + jnp.maximum(u1, 0.2 * u1) * attv[j1, :])
                _store_scalar(lbuf, (h, e), plsc.cumsum(t))
            return carry2

        lax.fori_loop(0, BLKA, edge, 0)
        for g in range(BLKA // 16):
            eids = g * 16 + _iota16()
            for h in range(4):
                ev = jnp.exp(lbuf[h, pl.ds(g * 16, 16)])
                plsc.store_scatter(e4, [eids, jnp.full((16,), h, _i32)], ev)
        pltpu.sync_copy(e4, expv1.at[pl.ds(base, BLKA), :])
        pltpu.sync_copy(e4, dsh.at[didx], add=True)
        return carry

    lax.fori_loop(0, CHUNKS_A, chunk, 0)
    plsc.subcore_barrier()
    pltpu.sync_copy(dsh.at[pl.ds(sid * ROWS_PER_SUB, ROWS_PER_SUB), :],
                    dpart.at[cid, pl.ds(sid * ROWS_PER_SUB, ROWS_PER_SUB), :])


def _l1a_call(t1, srcp, dstp, attr, z4):
    f = pl.kernel(
        _l1a_body,
        out_type=[jax.ShapeDtypeStruct((E_PAD, 4), _f32),
                  jax.ShapeDtypeStruct((2, NPAD, 4), _f32)],
        mesh=_mesh(),
        compiler_params=pltpu.CompilerParams(needs_layout_passes=False),
        scratch_types=[
            pltpu.VMEM((2 * BLKA,), _i32), pltpu.VMEM((BLKA,), _i32),
            pltpu.VMEM((2 * BLKA, 128), _f32),
            pltpu.VMEM((8, 16), _f32), pltpu.VMEM((4, BLKA), _f32),
            pltpu.VMEM((BLKA, 4), _f32),
            pltpu.VMEM_SHARED((NPAD, 4), _f32),
            pltpu.SemaphoreType.DMA,
        ],
    )
    return f(t1, srcp, dstp, attr, z4)


def _l1b_body(t1, srcp, dstp, expv1f, z128,
              opart,
              sidx, didx, xlr, expc, osh, sg1):
    cid = lax.axis_index("c")
    sid = lax.axis_index("s")
    pltpu.sync_copy(z128, osh.at[pl.ds(sid * ROWS_PER_SUB, ROWS_PER_SUB), :])
    plsc.subcore_barrier()
    w = cid * 16 + sid
    base_w = w * (CHUNKS * BLK)

    def chunk(c, carry):
        base = base_w + c * BLK
        pltpu.sync_copy(srcp.at[pl.ds(base, BLK)], sidx)
        pltpu.sync_copy(dstp.at[pl.ds(base, BLK)], didx)
        cp1 = pltpu.async_copy(t1.at[sidx], xlr, sg1)
        pltpu.sync_copy(expv1f.at[pl.ds(base * 4, BLK * 4)], expc)
        cp1.wait()

        def edge(e, carry2):
            e4 = e * 4
            for h in range(4):
                av = plsc.load_gather(expc, [jnp.full((16,), e4 + h, _i32)])
                for j in (2 * h, 2 * h + 1):
                    xlr[e, pl.ds(j * 16, 16)] = xlr[e, pl.ds(j * 16, 16)] * av
            return carry2

        lax.fori_loop(0, BLK, edge, 0)
        pltpu.sync_copy(xlr, osh.at[didx], add=True)
        return carry

    lax.fori_loop(0, CHUNKS, chunk, 0)
    plsc.subcore_barrier()
    pltpu.sync_copy(osh.at[pl.ds(sid * ROWS_PER_SUB, ROWS_PER_SUB), :],
                    opart.at[cid, pl.ds(sid * ROWS_PER_SUB, ROWS_PER_SUB), :])


def _l1b_call(t1, srcp, dstp, expv1f, z128):
    f = pl.kernel(
        _l1b_body,
        out_type=[jax.ShapeDtypeStruct((2, NPAD, 128), _f32)],
        mesh=_mesh(),
        compiler_params=pltpu.CompilerParams(needs_layout_passes=False),
        scratch_types=[
            pltpu.VMEM((BLK,), _i32), pltpu.VMEM((BLK,), _i32),
            pltpu.VMEM((BLK, 128), _f32), pltpu.VMEM((BLK * 4,), _f32),
            pltpu.VMEM_SHARED((NPAD, 128), _f32),
            pltpu.SemaphoreType.DMA,
        ],
    )
    return f(t1, srcp, dstp, expv1f, z128)[0]


def _l2a_body(t2, srcp, dstp, att2f, z4,
              expv2, dpart,
              cidx, didx, xlr, attv, lbuf, e4, dsh, sg1):
    cid = lax.axis_index("c")
    sid = lax.axis_index("s")
    pltpu.sync_copy(z4, dsh.at[pl.ds(sid * ROWS_PER_SUB, ROWS_PER_SUB), :])
    pltpu.sync_copy(att2f, attv)
    z16 = jnp.zeros((16,), _f32)
    for g in range(BLKA // 16):
        eids = g * 16 + _iota16()
        for h in range(1, 4):
            plsc.store_scatter(e4, [eids, jnp.full((16,), h, _i32)], z16)
    plsc.subcore_barrier()
    w = cid * 16 + sid
    base_w = w * (CHUNKS_A * BLKA)

    def chunk(c, carry):
        base = base_w + c * BLKA
        pltpu.sync_copy(srcp.at[pl.ds(base, BLKA)], cidx.at[pl.ds(0, BLKA)])
        pltpu.sync_copy(dstp.at[pl.ds(base, BLKA)], cidx.at[pl.ds(BLKA, BLKA)])
        pltpu.sync_copy(dstp.at[pl.ds(base, BLKA)], didx)
        for k in range(BLKA // 16):
            s = pl.ds(BLKA + k * 16, 16)
            cidx[s] = cidx[s] + NPAD
        pltpu.async_copy(t2.at[cidx], xlr, sg1).wait()
        a2 = attv[...]

        def edge(e, carry2):
            u = xlr[e, pl.ds(0, 16)] + xlr[BLKA + e, pl.ds(0, 16)]
            t = jnp.maximum(u, 0.2 * u) * a2
            _store_scalar(lbuf, (e,), plsc.cumsum(t))
            return carry2

        lax.fori_loop(0, BLKA, edge, 0)
        z0 = jnp.zeros((16,), _i32)
        for g in range(BLKA // 16):
            ev = jnp.exp(lbuf[pl.ds(g * 16, 16)])
            lbuf[pl.ds(g * 16, 16)] = ev
            plsc.store_scatter(e4, [g * 16 + _iota16(), z0], ev)
        pltpu.sync_copy(lbuf, expv2.at[pl.ds(base, BLKA)])
        pltpu.sync_copy(e4, dsh.at[didx], add=True)
        return carry

    lax.fori_loop(0, CHUNKS_A, chunk, 0)
    plsc.subcore_barrier()
    pltpu.sync_copy(dsh.at[pl.ds(sid * ROWS_PER_SUB, ROWS_PER_SUB), :],
                    dpart.at[cid, pl.ds(sid * ROWS_PER_SUB, ROWS_PER_SUB), :])


def _l2a_call(t2, srcp, dstp, att2f, z4):
    f = pl.kernel(
        _l2a_body,
        out_type=[jax.ShapeDtypeStruct((E_PAD,), _f32),
                  jax.ShapeDtypeStruct((2, NPAD, 4), _f32)],
        mesh=_mesh(),
        compiler_params=pltpu.CompilerParams(needs_layout_passes=False),
        scratch_types=[
            pltpu.VMEM((2 * BLKA,), _i32), pltpu.VMEM((BLKA,), _i32),
            pltpu.VMEM((2 * BLKA, 128), _f32),
            pltpu.VMEM((16,), _f32), pltpu.VMEM((BLKA,), _f32),
            pltpu.VMEM((BLKA, 4), _f32),
            pltpu.VMEM_SHARED((NPAD, 4), _f32),
            pltpu.SemaphoreType.DMA,
        ],
    )
    return f(t2, srcp, dstp, att2f, z4)


def _l2b_body(t2, srcp, dstp, expv2, z16r,
              opart,
              sidx, didx, xlr, rows16, ec, osh, sg1):
    cid = lax.axis_index("c")
    sid = lax.axis_index("s")
    pltpu.sync_copy(z16r, osh.at[pl.ds(sid * ROWS_PER_SUB, ROWS_PER_SUB), :])
    plsc.subcore_barrier()
    w = cid * 16 + sid
    base_w = w * (CHUNKS * BLK)

    def chunk(c, carry):
        base = base_w + c * BLK
        pltpu.sync_copy(srcp.at[pl.ds(base, BLK)], sidx)
        pltpu.sync_copy(dstp.at[pl.ds(base, BLK)], didx)
        cp1 = pltpu.async_copy(t2.at[sidx], xlr, sg1)
        pltpu.sync_copy(expv2.at[pl.ds(base, BLK)], ec)
        cp1.wait()

        def edge(e, carry2):
            av = plsc.load_gather(ec, [jnp.full((16,), e, _i32)])
            rows16[e, :] = xlr[e, pl.ds(0, 16)] * av
            return carry2

        lax.fori_loop(0, BLK, edge, 0)
        pltpu.sync_copy(rows16, osh.at[didx], add=True)
        return carry

    lax.fori_loop(0, CHUNKS, chunk, 0)
    plsc.subcore_barrier()
    pltpu.sync_copy(osh.at[pl.ds(sid * ROWS_PER_SUB, ROWS_PER_SUB), :],
                    opart.at[cid, pl.ds(sid * ROWS_PER_SUB, ROWS_PER_SUB), :])


def _l2b_call(t2, srcp, dstp, expv2, z16r):
    f = pl.kernel(
        _l2b_body,
        out_type=[jax.ShapeDtypeStruct((2, NPAD, 16), _f32)],
        mesh=_mesh(),
        compiler_params=pltpu.CompilerParams(needs_layout_passes=False),
        scratch_types=[
            pltpu.VMEM((BLK,), _i32), pltpu.VMEM((BLK,), _i32),
            pltpu.VMEM((BLK, 128), _f32), pltpu.VMEM((BLK, 16), _f32),
            pltpu.VMEM((BLK,), _f32),
            pltpu.VMEM_SHARED((NPAD, 16), _f32),
            pltpu.SemaphoreType.DMA,
        ],
    )
    return f(t2, srcp, dstp, expv2, z16r)[0]


def _mm1_k(x_ref, wl_ref, wr_ref, t_ref):
    xb = x_ref[...]
    t_ref[0] = jnp.dot(xb, wl_ref[...], preferred_element_type=_f32)
    t_ref[1] = jnp.dot(xb, wr_ref[...], preferred_element_type=_f32)


def _mm1(xpad, Wl1, Wr1):
    return pl.pallas_call(
        _mm1_k,
        grid=(20,),
        in_specs=[pl.BlockSpec((512, 128), lambda i: (i, 0)),
                  pl.BlockSpec((128, 128), lambda i: (0, 0)),
                  pl.BlockSpec((128, 128), lambda i: (0, 0))],
        out_specs=[pl.BlockSpec((2, 512, 128), lambda i: (0, i, 0))],
        out_shape=[jax.ShapeDtypeStruct((2, NPAD, 128), _f32)],
    )(xpad, Wl1, Wr1)[0]


def _mm2_k(op_ref, dp_ref, b1_ref, wl_ref, wr_ref, t_ref):
    i = pl.program_id(0)
    p = op_ref[0] + op_ref[1]
    den = dp_ref[0] + dp_ref[1]
    cols = []
    for h in range(4):
        cols.append(p[:, h * 32:(h + 1) * 32] / den[:, h:h + 1])
    p = jnp.concatenate(cols, axis=1) + b1_ref[...]
    rows = i * 512 + lax.broadcasted_iota(_i32, (512, 128), 0)
    hmat = jnp.where(rows < N, jnp.tanh(p), 0.0)
    t_ref[0] = jnp.dot(hmat, wl_ref[...], preferred_element_type=_f32)
    t_ref[1] = jnp.dot(hmat, wr_ref[...], preferred_element_type=_f32)


def _mm2(opart, dpart, b1, Wl2p, Wr2p):
    return pl.pallas_call(
        _mm2_k,
        grid=(20,),
        in_specs=[pl.BlockSpec((2, 512, 128), lambda i: (0, i, 0)),
                  pl.BlockSpec((2, 512, 4), lambda i: (0, i, 0)),
                  pl.BlockSpec((1, 128), lambda i: (0, 0)),
                  pl.BlockSpec((128, 128), lambda i: (0, 0)),
                  pl.BlockSpec((128, 128), lambda i: (0, 0))],
        out_specs=[pl.BlockSpec((2, 512, 128), lambda i: (0, i, 0))],
        out_shape=[jax.ShapeDtypeStruct((2, NPAD, 128), _f32)],
    )(opart, dpart, b1.reshape(1, 128), Wl2p, Wr2p)[0]


def _fin_k(op_ref, dp_ref, b2_ref, out_ref, ls_ref):
    den = dp_ref[0] + dp_ref[1]
    p = (op_ref[0] + op_ref[1]) / den[:, 0:1] + b2_ref[...]
    out_ref[...] = p
    m = jnp.max(p, axis=1, keepdims=True)
    s = jnp.log(jnp.sum(jnp.exp(p - m), axis=1, keepdims=True))
    ls_ref[...] = p - m - s


def _final(opart2, dpart2, b2):
    return pl.pallas_call(
        _fin_k,
        grid=(20,),
        in_specs=[pl.BlockSpec((2, 512, 16), lambda i: (0, i, 0)),
                  pl.BlockSpec((2, 512, 4), lambda i: (0, i, 0)),
                  pl.BlockSpec((1, 16), lambda i: (0, 0))],
        out_specs=[pl.BlockSpec((512, 16), lambda i: (i, 0)),
                   pl.BlockSpec((512, 16), lambda i: (i, 0))],
        out_shape=[jax.ShapeDtypeStruct((NPAD, 16), _f32),
                   jax.ShapeDtypeStruct((NPAD, 16), _f32)],
    )(opart2, dpart2, b2.reshape(1, 16))


def kernel(x, edge_index, Wl1, Wr1, att1, b1, Wl2, Wr2, att2, b2):
    xpad = jnp.zeros((NPAD, 128), _f32).at[:N].set(x)
    loop = jnp.arange(N, dtype=_i32)
    pad = jnp.full((E_PAD - E_TOT,), DUMMY, _i32)
    srcp = jnp.concatenate([edge_index[0].astype(_i32), loop, pad])
    dstp = jnp.concatenate([edge_index[1].astype(_i32), loop, pad])
    z4 = jnp.zeros((ROWS_PER_SUB, 4), _f32)
    z128 = jnp.zeros((ROWS_PER_SUB, 128), _f32)
    z16r = jnp.zeros((ROWS_PER_SUB, 16), _f32)
    att1r = att1.reshape(8, 16)
    att2f = att2.reshape(16)
    Wl2p = jnp.zeros((128, 128), _f32).at[:, :16].set(Wl2)
    Wr2p = jnp.zeros((128, 128), _f32).at[:, :16].set(Wr2)

    t1 = _mm1(xpad, Wl1, Wr1).reshape(2 * NPAD, 128)
    expv1, dpart1 = _l1a_call(t1, srcp, dstp, att1r, z4)
    opart1 = _l1b_call(t1, srcp, dstp, expv1.reshape(-1), z128)
    t2 = _mm2(opart1, dpart1, b1, Wl2p, Wr2p).reshape(2 * NPAD, 128)
    expv2, dpart2 = _l2a_call(t2, srcp, dstp, att2f, z4)
    opart2 = _l2b_call(t2, srcp, dstp, expv2, z16r)
    out, ls = _final(opart2, dpart2, b2)
    return (out[:N], ls[:N])


# speedup vs baseline: 20.8914x; 1.2925x over previous
import jax
import jax.numpy as jnp
from jax import lax
from jax.experimental import pallas as pl
from jax.experimental.pallas import tpu as pltpu
from jax.experimental.pallas import tpu_sc as plsc

N = 10000
NPAD = 10240
ROWS_PER_SUB = 640
E_RAW = 320000
E_TOT = E_RAW + N
BLK = 128
BLKA = 64
CHUNKS = 81
CHUNKS_A = 162
NW = 32
E_PAD = NW * CHUNKS * BLK
DUMMY = N

_f32 = jnp.float32
_i32 = jnp.int32


def _mesh():
    return plsc.VectorSubcoreMesh(
        core_axis_name="c", subcore_axis_name="s", num_cores=2, num_subcores=16)


def _iota16():
    return lax.iota(_i32, 16)


def _lane15():
    return lax.iota(_i32, 16) == 15


def _store_scalar(ref, idxs, vec):
    plsc.store_scatter(ref, [jnp.full((16,), i, _i32) for i in idxs], vec,
                       mask=_lane15())


def _l1_body(t1, srcp, dstp, attr, z4, z128,
             dpart, opart,
             cidx, didx, xlr, attv, lbuf, e4, dsh, osh, sg1):
    cid = lax.axis_index("c")
    sid = lax.axis_index("s")
    pltpu.sync_copy(z4, dsh.at[pl.ds(sid * ROWS_PER_SUB, ROWS_PER_SUB), :])
    pltpu.sync_copy(z128, osh.at[pl.ds(sid * ROWS_PER_SUB, ROWS_PER_SUB), :])
    pltpu.sync_copy(attr, attv)
    plsc.subcore_barrier()
    w = cid * 16 + sid
    base_w = w * (CHUNKS_A * BLKA)

    def chunk(c, carry):
        base = base_w + c * BLKA
        pltpu.sync_copy(srcp.at[pl.ds(base, BLKA)], cidx.at[pl.ds(0, BLKA)])
        pltpu.sync_copy(dstp.at[pl.ds(base, BLKA)], cidx.at[pl.ds(BLKA, BLKA)])
        pltpu.sync_copy(dstp.at[pl.ds(base, BLKA)], didx)
        for k in range(BLKA // 16):
            s = pl.ds(BLKA + k * 16, 16)
            cidx[s] = cidx[s] + NPAD
        pltpu.async_copy(t1.at[cidx], xlr, sg1).wait()

        def edge(e, carry2):
            for h in range(4):
                j0, j1 = 2 * h, 2 * h + 1
                u0 = xlr[e, pl.ds(j0 * 16, 16)] + xlr[BLKA + e, pl.ds(j0 * 16, 16)]
                u1 = xlr[e, pl.ds(j1 * 16, 16)] + xlr[BLKA + e, pl.ds(j1 * 16, 16)]
                t = (jnp.maximum(u0, 0.2 * u0) * attv[j0, :]
                     + jnp.maximum(u1, 0.2 * u1) * attv[j1, :])
                _store_scalar(lbuf, (h, e), plsc.cumsum(t))
            return carry2

        lax.fori_loop(0, BLKA, edge, 0)
        for g in range(BLKA // 16):
            eids = g * 16 + _iota16()
            for h in range(4):
                ev = jnp.exp(lbuf[h, pl.ds(g * 16, 16)])
                plsc.store_scatter(e4, [eids, jnp.full((16,), h, _i32)], ev)

        def edge2(e, carry2):
            ee = jnp.full((16,), e, _i32)
            for h in range(4):
                av = plsc.load_gather(e4, [ee, jnp.full((16,), h, _i32)])
                for j in (2 * h, 2 * h + 1):
                    xlr[e, pl.ds(j * 16, 16)] = xlr[e, pl.ds(j * 16, 16)] * av
            return carry2

        lax.fori_loop(0, BLKA, edge2, 0)
        pltpu.sync_copy(e4, dsh.at[didx], add=True)
        pltpu.sync_copy(xlr.at[pl.ds(0, BLKA), :], osh.at[didx], add=True)
        return carry

    lax.fori_loop(0, CHUNKS_A, chunk, 0)
    plsc.subcore_barrier()
    pltpu.sync_copy(dsh.at[pl.ds(sid * ROWS_PER_SUB, ROWS_PER_SUB), :],
                    dpart.at[cid, pl.ds(sid * ROWS_PER_SUB, ROWS_PER_SUB), :])
    pltpu.sync_copy(osh.at[pl.ds(sid * ROWS_PER_SUB, ROWS_PER_SUB), :],
                    opart.at[cid, pl.ds(sid * ROWS_PER_SUB, ROWS_PER_SUB), :])


def _l1_call(t1, srcp, dstp, attr, z4, z128):
    f = pl.kernel(
        _l1_body,
        out_type=[jax.ShapeDtypeStruct((2, NPAD, 4), _f32),
                  jax.ShapeDtypeStruct((2, NPAD, 128), _f32)],
        mesh=_mesh(),
        compiler_params=pltpu.CompilerParams(needs_layout_passes=False),
        scratch_types=[
            pltpu.VMEM((2 * BLKA,), _i32), pltpu.VMEM((BLKA,), _i32),
            pltpu.VMEM((2 * BLKA, 128), _f32),
            pltpu.VMEM((8, 16), _f32), pltpu.VMEM((4, BLKA), _f32),
            pltpu.VMEM((BLKA, 4), _f32),
            pltpu.VMEM_SHARED((NPAD, 4), _f32),
            pltpu.VMEM_SHARED((NPAD, 128), _f32),
            pltpu.SemaphoreType.DMA,
        ],
    )
    return f(t1, srcp, dstp, attr, z4, z128)


def _l2_body(t2, srcp, dstp, att2f, z4, z16r,
             dpart, opart,
             cidx, didx, xlr, rows16, attv, lbuf, e4, dsh, osh, sg1):
    cid = lax.axis_index("c")
    sid = lax.axis_index("s")
    pltpu.sync_copy(z4, dsh.at[pl.ds(sid * ROWS_PER_SUB, ROWS_PER_SUB), :])
    pltpu.sync_copy(z16r, osh.at[pl.ds(sid * ROWS_PER_SUB, ROWS_PER_SUB), :])
    pltpu.sync_copy(att2f, attv)
    z16 = jnp.zeros((16,), _f32)
    for g in range(BLKA // 16):
        eids = g * 16 + _iota16()
        for h in range(1, 4):
            plsc.store_scatter(e4, [eids, jnp.full((16,), h, _i32)], z16)
    plsc.subcore_barrier()
    w = cid * 16 + sid
    base_w = w * (CHUNKS_A * BLKA)

    def chunk(c, carry):
        base = base_w + c * BLKA
        pltpu.sync_copy(srcp.at[pl.ds(base, BLKA)], cidx.at[pl.ds(0, BLKA)])
        pltpu.sync_copy(dstp.at[pl.ds(base, BLKA)], cidx.at[pl.ds(BLKA, BLKA)])
        pltpu.sync_copy(dstp.at[pl.ds(base, BLKA)], didx)
        for k in range(BLKA // 16):
            s = pl.ds(BLKA + k * 16, 16)
            cidx[s] = cidx[s] + NPAD
        pltpu.async_copy(t2.at[cidx], xlr, sg1).wait()
        a2 = attv[...]

        def edge(e, carry2):
            u = xlr[e, pl.ds(0, 16)] + xlr[BLKA + e, pl.ds(0, 16)]
            t = jnp.maximum(u, 0.2 * u) * a2
            _store_scalar(lbuf, (e,), plsc.cumsum(t))
            return carry2

        lax.fori_loop(0, BLKA, edge, 0)
        z0 = jnp.zeros((16,), _i32)
        for g in range(BLKA // 16):
            ev = jnp.exp(lbuf[pl.ds(g * 16, 16)])
            plsc.store_scatter(e4, [g * 16 + _iota16(), z0], ev)

        def edge2(e, carry2):
            av = plsc.load_gather(e4, [jnp.full((16,), e, _i32),
                                       jnp.zeros((16,), _i32)])
            rows16[e, :] = xlr[e, pl.ds(0, 16)] * av
            return carry2

        lax.fori_loop(0, BLKA, edge2, 0)
        pltpu.sync_copy(e4, dsh.at[didx], add=True)
        pltpu.sync_copy(rows16, osh.at[didx], add=True)
        return carry

    lax.fori_loop(0, CHUNKS_A, chunk, 0)
    plsc.subcore_barrier()
    pltpu.sync_copy(dsh.at[pl.ds(sid * ROWS_PER_SUB, ROWS_PER_SUB), :],
                    dpart.at[cid, pl.ds(sid * ROWS_PER_SUB, ROWS_PER_SUB), :])
    pltpu.sync_copy(osh.at[pl.ds(sid * ROWS_PER_SUB, ROWS_PER_SUB), :],
                    opart.at[cid, pl.ds(sid * ROWS_PER_SUB, ROWS_PER_SUB), :])


def _l2_call(t2, srcp, dstp, att2f, z4, z16r):
    f = pl.kernel(
        _l2_body,
        out_type=[jax.ShapeDtypeStruct((2, NPAD, 4), _f32),
                  jax.ShapeDtypeStruct((2, NPAD, 16), _f32)],
        mesh=_mesh(),
        compiler_params=pltpu.CompilerParams(needs_layout_passes=False),
        scratch_types=[
            pltpu.VMEM((2 * BLKA,), _i32), pltpu.VMEM((BLKA,), _i32),
            pltpu.VMEM((2 * BLKA, 128), _f32), pltpu.VMEM((BLKA, 16), _f32),
            pltpu.VMEM((16,), _f32), pltpu.VMEM((BLKA,), _f32),
            pltpu.VMEM((BLKA, 4), _f32),
            pltpu.VMEM_SHARED((NPAD, 4), _f32),
            pltpu.VMEM_SHARED((NPAD, 16), _f32),
            pltpu.SemaphoreType.DMA,
        ],
    )
    return f(t2, srcp, dstp, att2f, z4, z16r)


def _mm1_k(x_ref, wl_ref, wr_ref, t_ref):
    xb = x_ref[...]
    t_ref[0] = jnp.dot(xb, wl_ref[...], preferred_element_type=_f32)
    t_ref[1] = jnp.dot(xb, wr_ref[...], preferred_element_type=_f32)


def _mm1(xpad, Wl1, Wr1):
    return pl.pallas_call(
        _mm1_k,
        grid=(20,),
        in_specs=[pl.BlockSpec((512, 128), lambda i: (i, 0)),
                  pl.BlockSpec((128, 128), lambda i: (0, 0)),
                  pl.BlockSpec((128, 128), lambda i: (0, 0))],
        out_specs=[pl.BlockSpec((2, 512, 128), lambda i: (0, i, 0))],
        out_shape=[jax.ShapeDtypeStruct((2, NPAD, 128), _f32)],
    )(xpad, Wl1, Wr1)[0]


def _mm2_k(op_ref, dp_ref, b1_ref, wl_ref, wr_ref, t_ref):
    i = pl.program_id(0)
    p = op_ref[0] + op_ref[1]
    den = dp_ref[0] + dp_ref[1]
    cols = []
    for h in range(4):
        cols.append(p[:, h * 32:(h + 1) * 32] / den[:, h:h + 1])
    p = jnp.concatenate(cols, axis=1) + b1_ref[...]
    rows = i * 512 + lax.broadcasted_iota(_i32, (512, 128), 0)
    hmat = jnp.where(rows < N, jnp.tanh(p), 0.0)
    t_ref[0] = jnp.dot(hmat, wl_ref[...], preferred_element_type=_f32)
    t_ref[1] = jnp.dot(hmat, wr_ref[...], preferred_element_type=_f32)


def _mm2(opart, dpart, b1, Wl2p, Wr2p):
    return pl.pallas_call(
        _mm2_k,
        grid=(20,),
        in_specs=[pl.BlockSpec((2, 512, 128), lambda i: (0, i, 0)),
                  pl.BlockSpec((2, 512, 4), lambda i: (0, i, 0)),
                  pl.BlockSpec((1, 128), lambda i: (0, 0)),
                  pl.BlockSpec((128, 128), lambda i: (0, 0)),
                  pl.BlockSpec((128, 128), lambda i: (0, 0))],
        out_specs=[pl.BlockSpec((2, 512, 128), lambda i: (0, i, 0))],
        out_shape=[jax.ShapeDtypeStruct((2, NPAD, 128), _f32)],
    )(opart, dpart, b1.reshape(1, 128), Wl2p, Wr2p)[0]


def _fin_k(op_ref, dp_ref, b2_ref, out_ref, ls_ref):
    den = dp_ref[0] + dp_ref[1]
    p = (op_ref[0] + op_ref[1]) / den[:, 0:1] + b2_ref[...]
    out_ref[...] = p
    m = jnp.max(p, axis=1, keepdims=True)
    s = jnp.log(jnp.sum(jnp.exp(p - m), axis=1, keepdims=True))
    ls_ref[...] = p - m - s


def _final(opart2, dpart2, b2):
    return pl.pallas_call(
        _fin_k,
        grid=(20,),
        in_specs=[pl.BlockSpec((2, 512, 16), lambda i: (0, i, 0)),
                  pl.BlockSpec((2, 512, 4), lambda i: (0, i, 0)),
                  pl.BlockSpec((1, 16), lambda i: (0, 0))],
        out_specs=[pl.BlockSpec((512, 16), lambda i: (i, 0)),
                   pl.BlockSpec((512, 16), lambda i: (i, 0))],
        out_shape=[jax.ShapeDtypeStruct((NPAD, 16), _f32),
                   jax.ShapeDtypeStruct((NPAD, 16), _f32)],
    )(opart2, dpart2, b2.reshape(1, 16))


def kernel(x, edge_index, Wl1, Wr1, att1, b1, Wl2, Wr2, att2, b2):
    xpad = jnp.zeros((NPAD, 128), _f32).at[:N].set(x)
    loop = jnp.arange(N, dtype=_i32)
    pad = jnp.full((E_PAD - E_TOT,), DUMMY, _i32)
    srcp = jnp.concatenate([edge_index[0].astype(_i32), loop, pad])
    dstp = jnp.concatenate([edge_index[1].astype(_i32), loop, pad])
    z4 = jnp.zeros((ROWS_PER_SUB, 4), _f32)
    z128 = jnp.zeros((ROWS_PER_SUB, 128), _f32)
    z16r = jnp.zeros((ROWS_PER_SUB, 16), _f32)
    att1r = att1.reshape(8, 16)
    att2f = att2.reshape(16)
    Wl2p = jnp.zeros((128, 128), _f32).at[:, :16].set(Wl2)
    Wr2p = jnp.zeros((128, 128), _f32).at[:, :16].set(Wr2)

    t1 = _mm1(xpad, Wl1, Wr1).reshape(2 * NPAD, 128)
    dpart1, opart1 = _l1_call(t1, srcp, dstp, att1r, z4, z128)
    t2 = _mm2(opart1, dpart1, b1, Wl2p, Wr2p).reshape(2 * NPAD, 128)
    dpart2, opart2 = _l2_call(t2, srcp, dstp, att2f, z4, z16r)
    out, ls = _final(opart2, dpart2, b2)
    return (out[:N], ls[:N])


# speedup vs baseline: 21.7606x; 1.0416x over previous
import jax
import jax.numpy as jnp
from jax import lax
from jax.experimental import pallas as pl
from jax.experimental.pallas import tpu as pltpu
from jax.experimental.pallas import tpu_sc as plsc

N = 10000
NPAD = 10240
ROWS_PER_SUB = 640
E_RAW = 320000
E_TOT = E_RAW + N
BLK = 128
BLKA = 64
CHUNKS = 81
CHUNKS_A = 162
NW = 32
E_PAD = NW * CHUNKS * BLK
DUMMY = N

_f32 = jnp.float32
_i32 = jnp.int32


def _mesh():
    return plsc.VectorSubcoreMesh(
        core_axis_name="c", subcore_axis_name="s", num_cores=2, num_subcores=16)


def _iota16():
    return lax.iota(_i32, 16)


def _lane15():
    return lax.iota(_i32, 16) == 15


def _store_scalar(ref, idxs, vec):
    plsc.store_scatter(ref, [jnp.full((16,), i, _i32) for i in idxs], vec,
                       mask=_lane15())


def _l1_body(t1, qidx, dstp, attr, z4, z128,
             dpart, opart,
             cidx, didx, xlr, attv, lbuf, e4, dsh, osh, sg1):
    cid = lax.axis_index("c")
    sid = lax.axis_index("s")
    pltpu.sync_copy(z4, dsh.at[pl.ds(sid * ROWS_PER_SUB, ROWS_PER_SUB), :])
    pltpu.sync_copy(z128, osh.at[pl.ds(sid * ROWS_PER_SUB, ROWS_PER_SUB), :])
    pltpu.sync_copy(attr, attv)
    plsc.subcore_barrier()
    w = cid * 16 + sid
    base_w = w * (CHUNKS_A * BLKA)

    def chunk(c, carry):
        base = base_w + c * BLKA
        pltpu.sync_copy(qidx.at[pl.ds(base * 2, 2 * BLKA)], cidx)
        pltpu.sync_copy(dstp.at[pl.ds(base, BLKA)], didx)
        pltpu.async_copy(t1.at[cidx], xlr, sg1).wait()
        o = 0

        def edge(e, carry2):
            for h in range(4):
                j0, j1 = 2 * h, 2 * h + 1
                u0 = xlr[o + e, pl.ds(j0 * 16, 16)] + xlr[o + BLKA + e, pl.ds(j0 * 16, 16)]
                u1 = xlr[o + e, pl.ds(j1 * 16, 16)] + xlr[o + BLKA + e, pl.ds(j1 * 16, 16)]
                t = (jnp.maximum(u0, 0.2 * u0) * attv[j0, :]
                     + jnp.maximum(u1, 0.2 * u1) * attv[j1, :])
                _store_scalar(lbuf, (h, e), plsc.cumsum(t))
            return carry2

        lax.fori_loop(0, BLKA, edge, 0, unroll=4)
        for g in range(BLKA // 16):
            eids = g * 16 + _iota16()
            for h in range(4):
                ev = jnp.exp(lbuf[h, pl.ds(g * 16, 16)])
                plsc.store_scatter(e4, [eids, jnp.full((16,), h, _i32)], ev)

        def edge2(e, carry2):
            ee = jnp.full((16,), e, _i32)
            for h in range(4):
                av = plsc.load_gather(e4, [ee, jnp.full((16,), h, _i32)])
                for j in (2 * h, 2 * h + 1):
                    xlr[o + e, pl.ds(j * 16, 16)] = xlr[o + e, pl.ds(j * 16, 16)] * av
            return carry2

        lax.fori_loop(0, BLKA, edge2, 0, unroll=4)
        pltpu.sync_copy(e4, dsh.at[didx], add=True)
        pltpu.sync_copy(xlr.at[pl.ds(0, BLKA), :], osh.at[didx], add=True)
        return carry

    lax.fori_loop(0, CHUNKS_A, chunk, 0)
    plsc.subcore_barrier()
    pltpu.sync_copy(dsh.at[pl.ds(sid * ROWS_PER_SUB, ROWS_PER_SUB), :],
                    dpart.at[cid, pl.ds(sid * ROWS_PER_SUB, ROWS_PER_SUB), :])
    pltpu.sync_copy(osh.at[pl.ds(sid * ROWS_PER_SUB, ROWS_PER_SUB), :],
                    opart.at[cid, pl.ds(sid * ROWS_PER_SUB, ROWS_PER_SUB), :])


def _l1_call(t1, qidx, dstp, attr, z4, z128):
    f = pl.kernel(
        _l1_body,
        out_type=[jax.ShapeDtypeStruct((2, NPAD, 4), _f32),
                  jax.ShapeDtypeStruct((2, NPAD, 128), _f32)],
        mesh=_mesh(),
        compiler_params=pltpu.CompilerParams(needs_layout_passes=False),
        scratch_types=[
            pltpu.VMEM((2 * BLKA,), _i32), pltpu.VMEM((BLKA,), _i32),
            pltpu.VMEM((2 * BLKA, 128), _f32),
            pltpu.VMEM((8, 16), _f32), pltpu.VMEM((4, BLKA), _f32),
            pltpu.VMEM((BLKA, 4), _f32),
            pltpu.VMEM_SHARED((NPAD, 4), _f32),
            pltpu.VMEM_SHARED((NPAD, 128), _f32),
            pltpu.SemaphoreType.DMA,
        ],
    )
    return f(t1, qidx, dstp, attr, z4, z128)


def _l2_body(t2, qidx, dstp, att2f, z4, z16r,
             dpart, opart,
             cidx, didx, xlr, rows16, attv, lbuf, e4, dsh, osh, sg1):
    cid = lax.axis_index("c")
    sid = lax.axis_index("s")
    pltpu.sync_copy(z4, dsh.at[pl.ds(sid * ROWS_PER_SUB, ROWS_PER_SUB), :])
    pltpu.sync_copy(z16r, osh.at[pl.ds(sid * ROWS_PER_SUB, ROWS_PER_SUB), :])
    pltpu.sync_copy(att2f, attv)
    z16 = jnp.zeros((16,), _f32)
    for g in range(BLKA // 16):
        eids = g * 16 + _iota16()
        for h in range(1, 4):
            plsc.store_scatter(e4, [eids, jnp.full((16,), h, _i32)], z16)
    plsc.subcore_barrier()
    w = cid * 16 + sid
    base_w = w * (CHUNKS_A * BLKA)

    def chunk(c, carry):
        base = base_w + c * BLKA
        pltpu.sync_copy(qidx.at[pl.ds(base * 2, 2 * BLKA)], cidx)
        pltpu.sync_copy(dstp.at[pl.ds(base, BLKA)], didx)
        pltpu.async_copy(t2.at[cidx], xlr, sg1).wait()
        o = 0
        a2 = attv[...]

        def edge(e, carry2):
            u = xlr[o + e, pl.ds(0, 16)] + xlr[o + BLKA + e, pl.ds(0, 16)]
            t = jnp.maximum(u, 0.2 * u) * a2
            _store_scalar(lbuf, (e,), plsc.cumsum(t))
            return carry2

        lax.fori_loop(0, BLKA, edge, 0, unroll=4)
        z0 = jnp.zeros((16,), _i32)
        for g in range(BLKA // 16):
            ev = jnp.exp(lbuf[pl.ds(g * 16, 16)])
            plsc.store_scatter(e4, [g * 16 + _iota16(), z0], ev)

        def edge2(e, carry2):
            av = plsc.load_gather(e4, [jnp.full((16,), e, _i32),
                                       jnp.zeros((16,), _i32)])
            rows16[e, :] = xlr[o + e, pl.ds(0, 16)] * av
            return carry2

        lax.fori_loop(0, BLKA, edge2, 0, unroll=4)
        pltpu.sync_copy(e4, dsh.at[didx], add=True)
        pltpu.sync_copy(rows16, osh.at[didx], add=True)
        return carry

    lax.fori_loop(0, CHUNKS_A, chunk, 0)
    plsc.subcore_barrier()
    pltpu.sync_copy(dsh.at[pl.ds(sid * ROWS_PER_SUB, ROWS_PER_SUB), :],
                    dpart.at[cid, pl.ds(sid * ROWS_PER_SUB, ROWS_PER_SUB), :])
    pltpu.sync_copy(osh.at[pl.ds(sid * ROWS_PER_SUB, ROWS_PER_SUB), :],
                    opart.at[cid, pl.ds(sid * ROWS_PER_SUB, ROWS_PER_SUB), :])


def _l2_call(t2, qidx, dstp, att2f, z4, z16r):
    f = pl.kernel(
        _l2_body,
        out_type=[jax.ShapeDtypeStruct((2, NPAD, 4), _f32),
                  jax.ShapeDtypeStruct((2, NPAD, 16), _f32)],
        mesh=_mesh(),
        compiler_params=pltpu.CompilerParams(needs_layout_passes=False),
        scratch_types=[
            pltpu.VMEM((2 * BLKA,), _i32), pltpu.VMEM((BLKA,), _i32),
            pltpu.VMEM((2 * BLKA, 128), _f32), pltpu.VMEM((BLKA, 16), _f32),
            pltpu.VMEM((16,), _f32), pltpu.VMEM((BLKA,), _f32),
            pltpu.VMEM((BLKA, 4), _f32),
            pltpu.VMEM_SHARED((NPAD, 4), _f32),
            pltpu.VMEM_SHARED((NPAD, 16), _f32),
            pltpu.SemaphoreType.DMA,
        ],
    )
    return f(t2, qidx, dstp, att2f, z4, z16r)


def _mm1_k(x_ref, wl_ref, wr_ref, t_ref):
    xb = x_ref[...]
    t_ref[0] = jnp.dot(xb, wl_ref[...], preferred_element_type=_f32)
    t_ref[1] = jnp.dot(xb, wr_ref[...], preferred_element_type=_f32)


def _mm1(xpad, Wl1, Wr1):
    return pl.pallas_call(
        _mm1_k,
        grid=(20,),
        in_specs=[pl.BlockSpec((512, 128), lambda i: (i, 0)),
                  pl.BlockSpec((128, 128), lambda i: (0, 0)),
                  pl.BlockSpec((128, 128), lambda i: (0, 0))],
        out_specs=[pl.BlockSpec((2, 512, 128), lambda i: (0, i, 0))],
        out_shape=[jax.ShapeDtypeStruct((2, NPAD, 128), _f32)],
    )(xpad, Wl1, Wr1)[0]


def _mm2_k(op_ref, dp_ref, b1_ref, wl_ref, wr_ref, t_ref):
    i = pl.program_id(0)
    p = op_ref[0] + op_ref[1]
    den = dp_ref[0] + dp_ref[1]
    cols = []
    for h in range(4):
        cols.append(p[:, h * 32:(h + 1) * 32] / den[:, h:h + 1])
    p = jnp.concatenate(cols, axis=1) + b1_ref[...]
    rows = i * 512 + lax.broadcasted_iota(_i32, (512, 128), 0)
    hmat = jnp.where(rows < N, jnp.tanh(p), 0.0)
    t_ref[0] = jnp.dot(hmat, wl_ref[...], preferred_element_type=_f32)
    t_ref[1] = jnp.dot(hmat, wr_ref[...], preferred_element_type=_f32)


def _mm2(opart, dpart, b1, Wl2p, Wr2p):
    return pl.pallas_call(
        _mm2_k,
        grid=(20,),
        in_specs=[pl.BlockSpec((2, 512, 128), lambda i: (0, i, 0)),
                  pl.BlockSpec((2, 512, 4), lambda i: (0, i, 0)),
                  pl.BlockSpec((1, 128), lambda i: (0, 0)),
                  pl.BlockSpec((128, 128), lambda i: (0, 0)),
                  pl.BlockSpec((128, 128), lambda i: (0, 0))],
        out_specs=[pl.BlockSpec((2, 512, 128), lambda i: (0, i, 0))],
        out_shape=[jax.ShapeDtypeStruct((2, NPAD, 128), _f32)],
    )(opart, dpart, b1.reshape(1, 128), Wl2p, Wr2p)[0]


def _fin_k(op_ref, dp_ref, b2_ref, out_ref, ls_ref):
    den = dp_ref[0] + dp_ref[1]
    p = (op_ref[0] + op_ref[1]) / den[:, 0:1] + b2_ref[...]
    out_ref[...] = p
    m = jnp.max(p, axis=1, keepdims=True)
    s = jnp.log(jnp.sum(jnp.exp(p - m), axis=1, keepdims=True))
    ls_ref[...] = p - m - s


def _final(opart2, dpart2, b2):
    return pl.pallas_call(
        _fin_k,
        grid=(20,),
        in_specs=[pl.BlockSpec((2, 512, 16), lambda i: (0, i, 0)),
                  pl.BlockSpec((2, 512, 4), lambda i: (0, i, 0)),
                  pl.BlockSpec((1, 16), lambda i: (0, 0))],
        out_specs=[pl.BlockSpec((512, 16), lambda i: (i, 0)),
                   pl.BlockSpec((512, 16), lambda i: (i, 0))],
        out_shape=[jax.ShapeDtypeStruct((NPAD, 16), _f32),
                   jax.ShapeDtypeStruct((NPAD, 16), _f32)],
    )(opart2, dpart2, b2.reshape(1, 16))


def kernel(x, edge_index, Wl1, Wr1, att1, b1, Wl2, Wr2, att2, b2):
    xpad = jnp.zeros((NPAD, 128), _f32).at[:N].set(x)
    loop = jnp.arange(N, dtype=_i32)
    pad = jnp.full((E_PAD - E_TOT,), DUMMY, _i32)
    srcp = jnp.concatenate([edge_index[0].astype(_i32), loop, pad])
    dstp = jnp.concatenate([edge_index[1].astype(_i32), loop, pad])
    qidx = jnp.concatenate([srcp.reshape(-1, BLKA),
                            dstp.reshape(-1, BLKA) + NPAD], axis=1).reshape(-1)
    z4 = jnp.zeros((ROWS_PER_SUB, 4), _f32)
    z128 = jnp.zeros((ROWS_PER_SUB, 128), _f32)
    z16r = jnp.zeros((ROWS_PER_SUB, 16), _f32)
    att1r = att1.reshape(8, 16)
    att2f = att2.reshape(16)
    Wl2p = jnp.zeros((128, 128), _f32).at[:, :16].set(Wl2)
    Wr2p = jnp.zeros((128, 128), _f32).at[:, :16].set(Wr2)
    t1 = _mm1(xpad, Wl1, Wr1).reshape(2 * NPAD, 128)
    dpart1, opart1 = _l1_call(t1, qidx, dstp, att1r, z4, z128)
    t2 = _mm2(opart1, dpart1, b1, Wl2p, Wr2p).reshape(2 * NPAD, 128)
    dpart2, opart2 = _l2_call(t2, qidx, dstp, att2f, z4, z16r)
    out, ls = _final(opart2, dpart2, b2)
    return (out[:N], ls[:N])


# speedup vs baseline: 21.9405x; 1.0083x over previous
import jax
import jax.numpy as jnp
from jax import lax
from jax.experimental import pallas as pl
from jax.experimental.pallas import tpu as pltpu
from jax.experimental.pallas import tpu_sc as plsc

N = 10000
NPAD = 10240
ROWS_PER_SUB = 640
E_RAW = 320000
E_TOT = E_RAW + N
BLK = 128
BLKA = 64
CHUNKS = 81
CHUNKS_A = 162
NW = 32
E_PAD = NW * CHUNKS * BLK
DUMMY = N

_f32 = jnp.float32
_i32 = jnp.int32


def _mesh():
    return plsc.VectorSubcoreMesh(
        core_axis_name="c", subcore_axis_name="s", num_cores=2, num_subcores=16)


def _iota16():
    return lax.iota(_i32, 16)


def _lane15():
    return lax.iota(_i32, 16) == 15


def _store_scalar(ref, idxs, vec):
    plsc.store_scatter(ref, [jnp.full((16,), i, _i32) for i in idxs], vec,
                       mask=_lane15())


def _l1_body(t1, qidx, dstp, attr, z4, z128,
             dpart, opart,
             cidx, didx, xlr, attv, lbuf, e4, dsh, osh, sg1):
    cid = lax.axis_index("c")
    sid = lax.axis_index("s")
    pltpu.sync_copy(z4, dsh.at[pl.ds(sid * ROWS_PER_SUB, ROWS_PER_SUB), :])
    pltpu.sync_copy(z128, osh.at[pl.ds(sid * ROWS_PER_SUB, ROWS_PER_SUB), :])
    pltpu.sync_copy(attr, attv)
    plsc.subcore_barrier()
    w = cid * 16 + sid
    base_w = w * (CHUNKS_A * BLKA)

    def chunk(c, carry):
        base = base_w + c * BLKA
        pltpu.sync_copy(qidx.at[pl.ds(base * 2, 2 * BLKA)], cidx)
        pltpu.sync_copy(dstp.at[pl.ds(base, BLKA)], didx)
        pltpu.async_copy(t1.at[cidx], xlr, sg1).wait()
        o = 0

        def edge(e, carry2):
            for h in range(4):
                j0, j1 = 2 * h, 2 * h + 1
                u0 = xlr[o + e, pl.ds(j0 * 16, 16)] + xlr[o + BLKA + e, pl.ds(j0 * 16, 16)]
                u1 = xlr[o + e, pl.ds(j1 * 16, 16)] + xlr[o + BLKA + e, pl.ds(j1 * 16, 16)]
                t = (jnp.maximum(u0, 0.2 * u0) * attv[j0, :]
                     + jnp.maximum(u1, 0.2 * u1) * attv[j1, :])
                _store_scalar(lbuf, (h, e), plsc.cumsum(t))
            return carry2

        lax.fori_loop(0, BLKA, edge, 0, unroll=4)
        for g in range(BLKA // 16):
            eids = g * 16 + _iota16()
            for h in range(4):
                ev = jnp.exp(lbuf[h, pl.ds(g * 16, 16)])
                plsc.store_scatter(e4, [eids, jnp.full((16,), h, _i32)], ev)

        def edge2(e, carry2):
            ee = jnp.full((16,), e, _i32)
            for h in range(4):
                av = plsc.load_gather(e4, [ee, jnp.full((16,), h, _i32)])
                for j in (2 * h, 2 * h + 1):
                    xlr[o + e, pl.ds(j * 16, 16)] = xlr[o + e, pl.ds(j * 16, 16)] * av
            return carry2

        lax.fori_loop(0, BLKA, edge2, 0, unroll=4)
        c1 = pltpu.async_copy(e4, dsh.at[didx], sg1, add=True)
        c2 = pltpu.async_copy(xlr.at[pl.ds(0, BLKA), :], osh.at[didx], sg1, add=True)
        c1.wait()
        c2.wait()
        return carry

    lax.fori_loop(0, CHUNKS_A, chunk, 0)
    plsc.subcore_barrier()
    pltpu.sync_copy(dsh.at[pl.ds(sid * ROWS_PER_SUB, ROWS_PER_SUB), :],
                    dpart.at[cid, pl.ds(sid * ROWS_PER_SUB, ROWS_PER_SUB), :])
    pltpu.sync_copy(osh.at[pl.ds(sid * ROWS_PER_SUB, ROWS_PER_SUB), :],
                    opart.at[cid, pl.ds(sid * ROWS_PER_SUB, ROWS_PER_SUB), :])


def _l1_call(t1, qidx, dstp, attr, z4, z128):
    f = pl.kernel(
        _l1_body,
        out_type=[jax.ShapeDtypeStruct((2, NPAD, 4), _f32),
                  jax.ShapeDtypeStruct((2, NPAD, 128), _f32)],
        mesh=_mesh(),
        compiler_params=pltpu.CompilerParams(needs_layout_passes=False),
        scratch_types=[
            pltpu.VMEM((2 * BLKA,), _i32), pltpu.VMEM((BLKA,), _i32),
            pltpu.VMEM((2 * BLKA, 128), _f32),
            pltpu.VMEM((8, 16), _f32), pltpu.VMEM((4, BLKA), _f32),
            pltpu.VMEM((BLKA, 4), _f32),
            pltpu.VMEM_SHARED((NPAD, 4), _f32),
            pltpu.VMEM_SHARED((NPAD, 128), _f32),
            pltpu.SemaphoreType.DMA,
        ],
    )
    return f(t1, qidx, dstp, attr, z4, z128)


def _l2_body(t2, qidx, dstp, att2f, z4, z16r,
             dpart, opart,
             cidx, didx, xlr, rows16, attv, lbuf, e4, dsh, osh, sg1):
    cid = lax.axis_index("c")
    sid = lax.axis_index("s")
    pltpu.sync_copy(z4, dsh.at[pl.ds(sid * ROWS_PER_SUB, ROWS_PER_SUB), :])
    pltpu.sync_copy(z16r, osh.at[pl.ds(sid * ROWS_PER_SUB, ROWS_PER_SUB), :])
    pltpu.sync_copy(att2f, attv)
    z16 = jnp.zeros((16,), _f32)
    for g in range(BLKA // 16):
        eids = g * 16 + _iota16()
        for h in range(1, 4):
            plsc.store_scatter(e4, [eids, jnp.full((16,), h, _i32)], z16)
    plsc.subcore_barrier()
    w = cid * 16 + sid
    base_w = w * (CHUNKS_A * BLKA)

    def chunk(c, carry):
        base = base_w + c * BLKA
        pltpu.sync_copy(qidx.at[pl.ds(base * 2, 2 * BLKA)], cidx)
        pltpu.sync_copy(dstp.at[pl.ds(base, BLKA)], didx)
        pltpu.async_copy(t2.at[cidx], xlr, sg1).wait()
        o = 0
        a2 = attv[...]

        def edge(e, carry2):
            u = xlr[o + e, pl.ds(0, 16)] + xlr[o + BLKA + e, pl.ds(0, 16)]
            t = jnp.maximum(u, 0.2 * u) * a2
            _store_scalar(lbuf, (e,), plsc.cumsum(t))
            return carry2

        lax.fori_loop(0, BLKA, edge, 0, unroll=4)
        z0 = jnp.zeros((16,), _i32)
        for g in range(BLKA // 16):
            ev = jnp.exp(lbuf[pl.ds(g * 16, 16)])
            plsc.store_scatter(e4, [g * 16 + _iota16(), z0], ev)

        def edge2(e, carry2):
            av = plsc.load_gather(e4, [jnp.full((16,), e, _i32),
                                       jnp.zeros((16,), _i32)])
            rows16[e, :] = xlr[o + e, pl.ds(0, 16)] * av
            return carry2

        lax.fori_loop(0, BLKA, edge2, 0, unroll=4)
        c1 = pltpu.async_copy(e4, dsh.at[didx], sg1, add=True)
        c2 = pltpu.async_copy(rows16, osh.at[didx], sg1, add=True)
        c1.wait()
        c2.wait()
        return carry

    lax.fori_loop(0, CHUNKS_A, chunk, 0)
    plsc.subcore_barrier()
    pltpu.sync_copy(dsh.at[pl.ds(sid * ROWS_PER_SUB, ROWS_PER_SUB), :],
                    dpart.at[cid, pl.ds(sid * ROWS_PER_SUB, ROWS_PER_SUB), :])
    pltpu.sync_copy(osh.at[pl.ds(sid * ROWS_PER_SUB, ROWS_PER_SUB), :],
                    opart.at[cid, pl.ds(sid * ROWS_PER_SUB, ROWS_PER_SUB), :])


def _l2_call(t2, qidx, dstp, att2f, z4, z16r):
    f = pl.kernel(
        _l2_body,
        out_type=[jax.ShapeDtypeStruct((2, NPAD, 4), _f32),
                  jax.ShapeDtypeStruct((2, NPAD, 16), _f32)],
        mesh=_mesh(),
        compiler_params=pltpu.CompilerParams(needs_layout_passes=False),
        scratch_types=[
            pltpu.VMEM((2 * BLKA,), _i32), pltpu.VMEM((BLKA,), _i32),
            pltpu.VMEM((2 * BLKA, 128), _f32), pltpu.VMEM((BLKA, 16), _f32),
            pltpu.VMEM((16,), _f32), pltpu.VMEM((BLKA,), _f32),
            pltpu.VMEM((BLKA, 4), _f32),
            pltpu.VMEM_SHARED((NPAD, 4), _f32),
            pltpu.VMEM_SHARED((NPAD, 16), _f32),
            pltpu.SemaphoreType.DMA,
        ],
    )
    return f(t2, qidx, dstp, att2f, z4, z16r)


def _mm1_k(x_ref, wl_ref, wr_ref, t_ref):
    xb = x_ref[...]
    t_ref[0] = jnp.dot(xb, wl_ref[...], preferred_element_type=_f32)
    t_ref[1] = jnp.dot(xb, wr_ref[...], preferred_element_type=_f32)


def _mm1(xpad, Wl1, Wr1):
    return pl.pallas_call(
        _mm1_k,
        grid=(20,),
        in_specs=[pl.BlockSpec((512, 128), lambda i: (i, 0)),
                  pl.BlockSpec((128, 128), lambda i: (0, 0)),
                  pl.BlockSpec((128, 128), lambda i: (0, 0))],
        out_specs=[pl.BlockSpec((2, 512, 128), lambda i: (0, i, 0))],
        out_shape=[jax.ShapeDtypeStruct((2, NPAD, 128), _f32)],
    )(xpad, Wl1, Wr1)[0]


def _mm2_k(op_ref, dp_ref, b1_ref, wl_ref, wr_ref, t_ref):
    i = pl.program_id(0)
    p = op_ref[0] + op_ref[1]
    den = dp_ref[0] + dp_ref[1]
    cols = []
    for h in range(4):
        cols.append(p[:, h * 32:(h + 1) * 32] / den[:, h:h + 1])
    p = jnp.concatenate(cols, axis=1) + b1_ref[...]
    rows = i * 512 + lax.broadcasted_iota(_i32, (512, 128), 0)
    hmat = jnp.where(rows < N, jnp.tanh(p), 0.0)
    t_ref[0] = jnp.dot(hmat, wl_ref[...], preferred_element_type=_f32)
    t_ref[1] = jnp.dot(hmat, wr_ref[...], preferred_element_type=_f32)


def _mm2(opart, dpart, b1, Wl2p, Wr2p):
    return pl.pallas_call(
        _mm2_k,
        grid=(20,),
        in_specs=[pl.BlockSpec((2, 512, 128), lambda i: (0, i, 0)),
                  pl.BlockSpec((2, 512, 4), lambda i: (0, i, 0)),
                  pl.BlockSpec((1, 128), lambda i: (0, 0)),
                  pl.BlockSpec((128, 128), lambda i: (0, 0)),
                  pl.BlockSpec((128, 128), lambda i: (0, 0))],
        out_specs=[pl.BlockSpec((2, 512, 128), lambda i: (0, i, 0))],
        out_shape=[jax.ShapeDtypeStruct((2, NPAD, 128), _f32)],
    )(opart, dpart, b1.reshape(1, 128), Wl2p, Wr2p)[0]


def _fin_k(op_ref, dp_ref, b2_ref, out_ref, ls_ref):
    den = dp_ref[0] + dp_ref[1]
    p = (op_ref[0] + op_ref[1]) / den[:, 0:1] + b2_ref[...]
    out_ref[...] = p
    m = jnp.max(p, axis=1, keepdims=True)
    s = jnp.log(jnp.sum(jnp.exp(p - m), axis=1, keepdims=True))
    ls_ref[...] = p - m - s


def _final(opart2, dpart2, b2):
    return pl.pallas_call(
        _fin_k,
        grid=(20,),
        in_specs=[pl.BlockSpec((2, 512, 16), lambda i: (0, i, 0)),
                  pl.BlockSpec((2, 512, 4), lambda i: (0, i, 0)),
                  pl.BlockSpec((1, 16), lambda i: (0, 0))],
        out_specs=[pl.BlockSpec((512, 16), lambda i: (i, 0)),
                   pl.BlockSpec((512, 16), lambda i: (i, 0))],
        out_shape=[jax.ShapeDtypeStruct((NPAD, 16), _f32),
                   jax.ShapeDtypeStruct((NPAD, 16), _f32)],
    )(opart2, dpart2, b2.reshape(1, 16))


def kernel(x, edge_index, Wl1, Wr1, att1, b1, Wl2, Wr2, att2, b2):
    xpad = jnp.zeros((NPAD, 128), _f32).at[:N].set(x)
    loop = jnp.arange(N, dtype=_i32)
    pad = jnp.full((E_PAD - E_TOT,), DUMMY, _i32)
    srcp = jnp.concatenate([edge_index[0].astype(_i32), loop, pad])
    dstp = jnp.concatenate([edge_index[1].astype(_i32), loop, pad])
    qidx = jnp.concatenate([srcp.reshape(-1, BLKA),
                            dstp.reshape(-1, BLKA) + NPAD], axis=1).reshape(-1)
    z4 = jnp.zeros((ROWS_PER_SUB, 4), _f32)
    z128 = jnp.zeros((ROWS_PER_SUB, 128), _f32)
    z16r = jnp.zeros((ROWS_PER_SUB, 16), _f32)
    att1r = att1.reshape(8, 16)
    att2f = att2.reshape(16)
    Wl2p = jnp.zeros((128, 128), _f32).at[:, :16].set(Wl2)
    Wr2p = jnp.zeros((128, 128), _f32).at[:, :16].set(Wr2)
    t1 = _mm1(xpad, Wl1, Wr1).reshape(2 * NPAD, 128)
    dpart1, opart1 = _l1_call(t1, qidx, dstp, att1r, z4, z128)
    t2 = _mm2(opart1, dpart1, b1, Wl2p, Wr2p).reshape(2 * NPAD, 128)
    dpart2, opart2 = _l2_call(t2, qidx, dstp, att2f, z4, z16r)
    out, ls = _final(opart2, dpart2, b2)
    return (out[:N], ls[:N])
